# baseline (speedup 1.0000x reference)
import functools

import jax
import jax.numpy as jnp
from jax import lax
from jax.experimental import pallas as pl
from jax.experimental.pallas import tpu as pltpu
from jax.experimental.pallas import tpu_sc as plsc

_F32 = jnp.float32
_NW = 32
_K = 128
_NBUF = 3


def _w(shape):
    nd = len(shape)
    return pl.BlockSpec(shape, lambda i, _nd=nd: (0,) * nd)


def _row(blk, d):
    return pl.BlockSpec((blk, d), lambda i: (i, 0))


def _tc_edge_enc(ea, V0, vb0, G0, g0, G1, g1):
    E = ea.shape[0]
    BLK = 2000
    def body(ear, V0r, vb0r, G0r, g0r, G1r, g1r, c0r, c1r):
        z = jnp.dot(ear[...], V0r[...], preferred_element_type=_F32) + vb0r[...]
        z = z * jax.nn.sigmoid(z)
        c0r[...] = jnp.dot(z, G0r[...], preferred_element_type=_F32) + g0r[...]
        c1r[...] = jnp.dot(z, G1r[...], preferred_element_type=_F32) + g1r[...]
    return pl.pallas_call(
        body,
        grid=(E // BLK,),
        in_specs=[_row(BLK, 4), _w((4, 32)), _w((1, 32)), _w((32, 32)),
                  _w((1, 32)), _w((32, 32)), _w((1, 32))],
        out_specs=[_row(BLK, 32), _row(BLK, 32)],
        out_shape=[jax.ShapeDtypeStruct((E, 32), _F32)] * 2,
    )(ea, V0, vb0, G0, g0, G1, g1)


def _tc_node_enc(x2, W0, b0, W1, b1, As, Ad):
    Nn = x2.shape[0]
    BLK = 2000
    def body(xr, W0r, b0r, W1r, b1r, Asr, Adr, hr, hsr, hdr):
        z = jnp.dot(xr[...], W0r[...], preferred_element_type=_F32) + b0r[...]
        z = z * jax.nn.sigmoid(z)
        h = jnp.dot(z, W1r[...], preferred_element_type=_F32) + b1r[...]
        hr[...] = h
        hsr[...] = jnp.dot(h, Asr[...], preferred_element_type=_F32)
        hdr[...] = jnp.dot(h, Adr[...], preferred_element_type=_F32)
    return pl.pallas_call(
        body,
        grid=(Nn // BLK,),
        in_specs=[_row(BLK, 16), _w((16, 32)), _w((1, 32)), _w((32, 32)),
                  _w((1, 32)), _w((32, 32)), _w((32, 32))],
        out_specs=[_row(BLK, 32)] * 3,
        out_shape=[jax.ShapeDtypeStruct((Nn, 32), _F32)] * 3,
    )(x2, W0, b0, W1, b1, As, Ad)


def _tc_cupdate(c1, t0, M):
    E = c1.shape[0]
    BLK = 2000
    def body(c1r, t0r, Mr, outr):
        outr[...] = c1r[...] + jnp.dot(t0r[...], Mr[...], preferred_element_type=_F32)
    return pl.pallas_call(
        body,
        grid=(E // BLK,),
        in_specs=[_row(BLK, 32), _row(BLK, 32), _w((32, 32))],
        out_specs=_row(BLK, 32),
        out_shape=jax.ShapeDtypeStruct((E, 32), _F32),
    )(c1, t0, M)


def _tc_node_update(h, Sa, Sb, dga, dgb, B, d, Ph, Pa, p, Q, q, As, Ad):
    Nn = h.shape[0]
    BLK = 2000
    def body(hr, Sar, Sbr, dgar, dgbr, Br, dr, Phr, Par, pr, Qr, qr, Asr, Adr,
             h1r, hsr, hdr):
        deg = dgar[:, 0:1] + dgbr[:, 0:1]
        agg = jnp.dot(Sar[...] + Sbr[...], Br[...], preferred_element_type=_F32) \
            + deg * dr[...]
        u = jnp.dot(hr[...], Phr[...], preferred_element_type=_F32) \
            + jnp.dot(agg, Par[...], preferred_element_type=_F32) + pr[...]
        u = u * jax.nn.sigmoid(u)
        h1 = hr[...] + jnp.dot(u, Qr[...], preferred_element_type=_F32) + qr[...]
        h1r[...] = h1
        hsr[...] = jnp.dot(h1, Asr[...], preferred_element_type=_F32)
        hdr[...] = jnp.dot(h1, Adr[...], preferred_element_type=_F32)
    return pl.pallas_call(
        body,
        grid=(Nn // BLK,),
        in_specs=[_row(BLK, 32), _row(BLK, 32), _row(BLK, 32), _row(BLK, 4),
                  _row(BLK, 4), _w((32, 32)), _w((1, 32)), _w((32, 32)),
                  _w((32, 32)), _w((1, 32)), _w((32, 32)), _w((1, 32)),
                  _w((32, 32)), _w((32, 32))],
        out_specs=[_row(BLK, 32)] * 3,
        out_shape=[jax.ShapeDtypeStruct((Nn, 32), _F32)] * 3,
    )(h, Sa, Sb, dga, dgb, B, d, Ph, Pa, p, Q, q, As, Ad)


def _tc_node_dec(h, Sa, Sb, dga, dgb, B, d, Ph, Pa, p, Q, q, D0, e0, D1, e1):
    Nn = h.shape[0]
    BLK = 2000
    def body(hr, Sar, Sbr, dgar, dgbr, Br, dr, Phr, Par, pr, Qr, qr,
             D0r, e0r, D1r, e1r, yr):
        deg = dgar[:, 0:1] + dgbr[:, 0:1]
        agg = jnp.dot(Sar[...] + Sbr[...], Br[...], preferred_element_type=_F32) \
            + deg * dr[...]
        u = jnp.dot(hr[...], Phr[...], preferred_element_type=_F32) \
            + jnp.dot(agg, Par[...], preferred_element_type=_F32) + pr[...]
        u = u * jax.nn.sigmoid(u)
        h2 = hr[...] + jnp.dot(u, Qr[...], preferred_element_type=_F32) + qr[...]
        z = jnp.dot(h2, D0r[...], preferred_element_type=_F32) + e0r[...]
        z = z * jax.nn.sigmoid(z)
        yr[...] = jnp.dot(z, D1r[...], preferred_element_type=_F32) + e1r[...]
    return pl.pallas_call(
        body,
        grid=(Nn // BLK,),
        in_specs=[_row(BLK, 32), _row(BLK, 32), _row(BLK, 32), _row(BLK, 4),
                  _row(BLK, 4), _w((32, 32)), _w((1, 32)), _w((32, 32)),
                  _w((32, 32)), _w((1, 32)), _w((32, 32)), _w((1, 32)),
                  _w((32, 32)), _w((1, 32)), _w((32, 16)), _w((1, 16))],
        out_specs=_row(BLK, 16),
        out_shape=jax.ShapeDtypeStruct((Nn, 16), _F32),
    )(h, Sa, Sb, dga, dgb, B, d, Ph, Pa, p, Q, q, D0, e0, D1, e1)


def _sc_edge_step(hs, hd, src2, dst2, c, z32, emit_t):
    E = c.shape[0]
    Nn = z32.shape[0]
    KB = src2.shape[1]
    NB = E // KB
    NBT = src2.shape[0] // _NW
    nbuf = 2 if emit_t else 3
    mesh = plsc.VectorSubcoreMesh(core_axis_name="c", subcore_axis_name="s")
    nper = 6 if emit_t else 5
    out_type = (jax.ShapeDtypeStruct((2, Nn, 32), _F32),)
    if emit_t:
        out_type = out_type + (jax.ShapeDtypeStruct((E, 32), _F32),)

    scratch = []
    for _s in range(nbuf):
        scratch += [pltpu.VMEM((KB,), jnp.int32)] * 2
        scratch += [pltpu.VMEM((KB, 32), _F32)] * (nper - 2)
    scratch += [pltpu.VMEM_SHARED((Nn, 32), _F32)]
    scratch += [pltpu.SemaphoreType.DMA] * (3 * nbuf)

    @functools.partial(
        pl.kernel,
        mesh=mesh,
        compiler_params=pltpu.CompilerParams(use_tc_tiling_on_sc=False),
        out_type=out_type,
        scratch_types=scratch,
    )
    def k(hs_hbm, hd_hbm, src_hbm, dst_hbm, c_hbm, z32_hbm, *rest):
        if emit_t:
            S_hbm, t_hbm = rest[0], rest[1]
            rs = rest[2:]
        else:
            S_hbm = rest[0]
            rs = rest[1:]
        slots = [rs[i * nper:(i + 1) * nper] for i in range(nbuf)]
        table = rs[nbuf * nper]
        sems = rs[nbuf * nper + 1:]
        lsem = sems[0:nbuf]
        gsem = sems[nbuf:2 * nbuf]
        osem = sems[2 * nbuf:3 * nbuf]
        cid = lax.axis_index("c")
        sid = lax.axis_index("s")
        wid = sid * 2 + cid
        nb = jnp.minimum(NBT, NB - wid * NBT)
        ebase = wid * (NBT * KB)

        @pl.when(sid == 0)
        def _init():
            pltpu.sync_copy(z32_hbm, table)

        plsc.subcore_barrier()

        def stage_a(j, s):
            @pl.when(j < nb)
            def _():
                sl = slots[s]
                pltpu.async_copy(src_hbm.at[wid * NBT + j], sl[0], lsem[s])
                pltpu.async_copy(dst_hbm.at[wid * NBT + j], sl[1], lsem[s])
                pltpu.async_copy(c_hbm.at[pl.ds(ebase + j * KB, KB)], sl[4], lsem[s])

        def stage_b(j, s):
            @pl.when(j < nb)
            def _():
                sl = slots[s]
                pltpu.make_async_copy(src_hbm.at[0], sl[0], lsem[s]).wait()
                pltpu.make_async_copy(src_hbm.at[0], sl[1], lsem[s]).wait()
                pltpu.make_async_copy(c_hbm.at[pl.ds(0, KB)], sl[4], lsem[s]).wait()
                pltpu.async_copy(hs_hbm.at[sl[0]], sl[2], gsem[s])
                pltpu.async_copy(hd_hbm.at[sl[1]], sl[3], gsem[s])

        def stage_c(j, s):
            @pl.when(j < nb)
            def _():
                sl = slots[s]
                pltpu.make_async_copy(c_hbm.at[pl.ds(0, KB)], sl[2], gsem[s]).wait()
                pltpu.make_async_copy(c_hbm.at[pl.ds(0, KB)], sl[3], gsem[s]).wait()
                if emit_t:
                    @pl.when(j >= nbuf)
                    def _w():
                        pltpu.make_async_copy(c_hbm.at[pl.ds(0, KB)], sl[5], osem[s]).wait()
                dstbuf = sl[5] if emit_t else sl[4]

                def row(r, carry2):
                    for half in range(2):
                        cs = pl.ds(half * 16, 16)
                        v = sl[4][r, cs] + sl[2][r, cs] + sl[3][r, cs]
                        dstbuf[r, cs] = v / (1.0 + jnp.exp(-v))
                    return carry2

                lax.fori_loop(0, KB, row, 0, unroll=4)
                pltpu.sync_copy(dstbuf, table.at[sl[1]], add=True)
                if emit_t:
                    pltpu.async_copy(dstbuf, t_hbm.at[pl.ds(ebase + j * KB, KB)],
                                     osem[s])

        look = nbuf - 1
        for s in range(nbuf):
            stage_a(s, s)
        for l in range(look):
            stage_b(l, l % nbuf)

        def outer(g, carry):
            for s in range(nbuf):
                j = g * nbuf + s
                stage_b(j + look, (s + look) % nbuf)
                stage_c(j, s)
                stage_a(j + nbuf, s)
            return carry

        lax.fori_loop(0, -(-NBT // nbuf), outer, 0)
        if emit_t:
            for s in range(nbuf):
                pltpu.make_async_copy(c_hbm.at[pl.ds(0, KB)], slots[s][5],
                                      osem[s]).wait()
        plsc.subcore_barrier()

        @pl.when(sid == 0)
        def _out():
            pltpu.sync_copy(table, S_hbm.at[cid])

    return k(hs, hd, src2, dst2, c, z32)


def _sc_degree(dst2, z4, o4, E):
    Nn = z4.shape[0]
    KB = dst2.shape[1]
    NB = E // KB
    NBT = dst2.shape[0] // _NW
    mesh = plsc.VectorSubcoreMesh(core_axis_name="c", subcore_axis_name="s")

    @functools.partial(
        pl.kernel,
        mesh=mesh,
        compiler_params=pltpu.CompilerParams(use_tc_tiling_on_sc=False),
        out_type=jax.ShapeDtypeStruct((2, Nn, 4), _F32),
        scratch_types=[
            pltpu.VMEM((NBT, KB), jnp.int32),
            pltpu.VMEM((KB, 4), _F32),
            pltpu.VMEM_SHARED((Nn, 4), _F32),
        ],
    )
    def k(dst_hbm, z4_hbm, o4_hbm, D_hbm, diAll, onev, degt):
        cid = lax.axis_index("c")
        sid = lax.axis_index("s")
        wid = sid * 2 + cid
        nb = jnp.minimum(NBT, NB - wid * NBT)

        @pl.when(sid == 0)
        def _init():
            pltpu.sync_copy(z4_hbm, degt)

        pltpu.sync_copy(o4_hbm, onev)
        pltpu.sync_copy(dst_hbm.at[pl.ds(wid * NBT, NBT)], diAll)
        plsc.subcore_barrier()

        def blk(j, carry):
            pltpu.sync_copy(onev, degt.at[diAll.at[j]], add=True)
            return carry

        lax.fori_loop(0, nb, blk, 0)
        plsc.subcore_barrier()

        @pl.when(sid == 0)
        def _out():
            pltpu.sync_copy(degt, D_hbm.at[cid])

    return k(dst2, z4, o4)


def kernel(x, edge_index, edge_attr, params):
    x2 = x[0]
    ea2 = edge_attr[0]
    src = edge_index[0]
    dst = edge_index[1]
    Nn = x2.shape[0]

    (W0, b0), (W1, b1) = params["enc_node"]
    (V0, vb0), (V1, vb1) = params["enc_edge"]
    (D0, e0), (D1, e1) = params["dec"]
    st = []
    for s in range(2):
        (A1, a1), (B1, d1) = params["steps"][s]["edge"]
        (P1, p1), (Q1, q1) = params["steps"][s]["node"]
        st.append(dict(Ae=A1[0:32], As=A1[32:64], Ad=A1[64:96], a=a1, B=B1,
                       d=d1, Ph=P1[0:32], Pa=P1[32:64], p=p1, Q=Q1, q=q1))

    r1 = lambda v: v.reshape(1, -1)
    mm = functools.partial(jnp.matmul)
    G0 = mm(V1, st[0]["Ae"])
    g0 = mm(vb1, st[0]["Ae"]) + st[0]["a"]
    G1 = mm(V1, st[1]["Ae"])
    g1 = mm(vb1 + st[0]["d"], st[1]["Ae"]) + st[1]["a"]
    M = mm(st[0]["B"], st[1]["Ae"])

    c0, c1 = _tc_edge_enc(ea2, V0, r1(vb0), G0, r1(g0), G1, r1(g1))
    h0, hs0, hd0 = _tc_node_enc(x2, W0, r1(b0), W1, r1(b1), st[0]["As"], st[0]["Ad"])

    z32 = jnp.zeros((Nn, 32), _F32)
    z4 = jnp.zeros((Nn, 4), _F32)

    KB = 80
    E = src.shape[0]
    NB = E // KB
    NBT = -(-NB // _NW)
    padlen = _NW * NBT * KB - E
    zpad = jnp.zeros((padlen,), jnp.int32)
    src2 = jnp.concatenate([src, zpad]).reshape(_NW * NBT, KB)
    dst2 = jnp.concatenate([dst, zpad]).reshape(_NW * NBT, KB)
    o4 = jnp.ones((KB, 4), _F32)

    Dp0 = _sc_degree(dst2, z4, o4, E)
    Sp0, t0 = _sc_edge_step(hs0, hd0, src2, dst2, c0, z32, True)
    c1p = _tc_cupdate(c1, t0, M)
    h1, hs1, hd1 = _tc_node_update(
        h0, Sp0[0], Sp0[1], Dp0[0], Dp0[1], st[0]["B"], r1(st[0]["d"]),
        st[0]["Ph"], st[0]["Pa"], r1(st[0]["p"]), st[0]["Q"], r1(st[0]["q"]),
        st[1]["As"], st[1]["Ad"])

    (Sp1,) = _sc_edge_step(hs1, hd1, src2, dst2, c1p, z32, False)
    y = _tc_node_dec(
        h1, Sp1[0], Sp1[1], Dp0[0], Dp0[1], st[1]["B"], r1(st[1]["d"]),
        st[1]["Ph"], st[1]["Pa"], r1(st[1]["p"]), st[1]["Q"], r1(st[1]["q"]),
        D0, r1(e0), D1, r1(e1))
    return y[None]


# speedup vs baseline: 1.0447x; 1.0447x over previous
import functools

import jax
import jax.numpy as jnp
from jax import lax
from jax.experimental import pallas as pl
from jax.experimental.pallas import tpu as pltpu
from jax.experimental.pallas import tpu_sc as plsc

_F32 = jnp.float32
_NW = 32
_K = 128
_NBUF = 3


def _w(shape):
    nd = len(shape)
    return pl.BlockSpec(shape, lambda i, _nd=nd: (0,) * nd)


def _row(blk, d):
    return pl.BlockSpec((blk, d), lambda i: (i, 0))


def _tc_edge_enc(ea, V0, vb0, G0, g0, G1, g1):
    E = ea.shape[0]
    BLK = 2000
    def body(ear, V0r, vb0r, G0r, g0r, G1r, g1r, c0r, c1r):
        z = jnp.dot(ear[...], V0r[...], preferred_element_type=_F32) + vb0r[...]
        z = z * jax.nn.sigmoid(z)
        c0r[...] = jnp.dot(z, G0r[...], preferred_element_type=_F32) + g0r[...]
        c1r[...] = jnp.dot(z, G1r[...], preferred_element_type=_F32) + g1r[...]
    return pl.pallas_call(
        body,
        grid=(E // BLK,),
        in_specs=[_row(BLK, 4), _w((4, 32)), _w((1, 32)), _w((32, 32)),
                  _w((1, 32)), _w((32, 32)), _w((1, 32))],
        out_specs=[_row(BLK, 32), _row(BLK, 32)],
        out_shape=[jax.ShapeDtypeStruct((E, 32), _F32)] * 2,
    )(ea, V0, vb0, G0, g0, G1, g1)


def _tc_node_enc(x2, W0, b0, W1, b1, As, Ad):
    Nn = x2.shape[0]
    BLK = 2000
    def body(xr, W0r, b0r, W1r, b1r, Asr, Adr, hr, hsr, hdr):
        z = jnp.dot(xr[...], W0r[...], preferred_element_type=_F32) + b0r[...]
        z = z * jax.nn.sigmoid(z)
        h = jnp.dot(z, W1r[...], preferred_element_type=_F32) + b1r[...]
        hr[...] = h
        hsr[...] = jnp.dot(h, Asr[...], preferred_element_type=_F32)
        hdr[...] = jnp.dot(h, Adr[...], preferred_element_type=_F32)
    return pl.pallas_call(
        body,
        grid=(Nn // BLK,),
        in_specs=[_row(BLK, 16), _w((16, 32)), _w((1, 32)), _w((32, 32)),
                  _w((1, 32)), _w((32, 32)), _w((32, 32))],
        out_specs=[_row(BLK, 32)] * 3,
        out_shape=[jax.ShapeDtypeStruct((Nn, 32), _F32)] * 3,
    )(x2, W0, b0, W1, b1, As, Ad)


def _tc_cupdate(c1, t0, M):
    E = c1.shape[0]
    BLK = 2000
    def body(c1r, t0r, Mr, outr):
        outr[...] = c1r[...] + jnp.dot(t0r[...], Mr[...], preferred_element_type=_F32)
    return pl.pallas_call(
        body,
        grid=(E // BLK,),
        in_specs=[_row(BLK, 32), _row(BLK, 32), _w((32, 32))],
        out_specs=_row(BLK, 32),
        out_shape=jax.ShapeDtypeStruct((E, 32), _F32),
    )(c1, t0, M)


def _tc_node_update(h, Sa, Sb, dga, dgb, B, d, Ph, Pa, p, Q, q, As, Ad):
    Nn = h.shape[0]
    BLK = 2000
    def body(hr, Sar, Sbr, dgar, dgbr, Br, dr, Phr, Par, pr, Qr, qr, Asr, Adr,
             h1r, hsr, hdr):
        deg = dgar[:, 0:1] + dgbr[:, 0:1]
        agg = jnp.dot(Sar[...] + Sbr[...], Br[...], preferred_element_type=_F32) \
            + deg * dr[...]
        u = jnp.dot(hr[...], Phr[...], preferred_element_type=_F32) \
            + jnp.dot(agg, Par[...], preferred_element_type=_F32) + pr[...]
        u = u * jax.nn.sigmoid(u)
        h1 = hr[...] + jnp.dot(u, Qr[...], preferred_element_type=_F32) + qr[...]
        h1r[...] = h1
        hsr[...] = jnp.dot(h1, Asr[...], preferred_element_type=_F32)
        hdr[...] = jnp.dot(h1, Adr[...], preferred_element_type=_F32)
    return pl.pallas_call(
        body,
        grid=(Nn // BLK,),
        in_specs=[_row(BLK, 32), _row(BLK, 32), _row(BLK, 32), _row(BLK, 4),
                  _row(BLK, 4), _w((32, 32)), _w((1, 32)), _w((32, 32)),
                  _w((32, 32)), _w((1, 32)), _w((32, 32)), _w((1, 32)),
                  _w((32, 32)), _w((32, 32))],
        out_specs=[_row(BLK, 32)] * 3,
        out_shape=[jax.ShapeDtypeStruct((Nn, 32), _F32)] * 3,
    )(h, Sa, Sb, dga, dgb, B, d, Ph, Pa, p, Q, q, As, Ad)


def _tc_node_dec(h, Sa, Sb, dga, dgb, B, d, Ph, Pa, p, Q, q, D0, e0, D1, e1):
    Nn = h.shape[0]
    BLK = 2000
    def body(hr, Sar, Sbr, dgar, dgbr, Br, dr, Phr, Par, pr, Qr, qr,
             D0r, e0r, D1r, e1r, yr):
        deg = dgar[:, 0:1] + dgbr[:, 0:1]
        agg = jnp.dot(Sar[...] + Sbr[...], Br[...], preferred_element_type=_F32) \
            + deg * dr[...]
        u = jnp.dot(hr[...], Phr[...], preferred_element_type=_F32) \
            + jnp.dot(agg, Par[...], preferred_element_type=_F32) + pr[...]
        u = u * jax.nn.sigmoid(u)
        h2 = hr[...] + jnp.dot(u, Qr[...], preferred_element_type=_F32) + qr[...]
        z = jnp.dot(h2, D0r[...], preferred_element_type=_F32) + e0r[...]
        z = z * jax.nn.sigmoid(z)
        yr[...] = jnp.dot(z, D1r[...], preferred_element_type=_F32) + e1r[...]
    return pl.pallas_call(
        body,
        grid=(Nn // BLK,),
        in_specs=[_row(BLK, 32), _row(BLK, 32), _row(BLK, 32), _row(BLK, 4),
                  _row(BLK, 4), _w((32, 32)), _w((1, 32)), _w((32, 32)),
                  _w((32, 32)), _w((1, 32)), _w((32, 32)), _w((1, 32)),
                  _w((32, 32)), _w((1, 32)), _w((32, 16)), _w((1, 16))],
        out_specs=_row(BLK, 16),
        out_shape=jax.ShapeDtypeStruct((Nn, 16), _F32),
    )(h, Sa, Sb, dga, dgb, B, d, Ph, Pa, p, Q, q, D0, e0, D1, e1)


def _sc_gather_silu(hs, hd, src2, dst2, c):
    E = c.shape[0]
    KB = src2.shape[1]
    NB = E // KB
    NBT = src2.shape[0] // _NW
    mesh = plsc.VectorSubcoreMesh(core_axis_name="c", subcore_axis_name="s")

    @functools.partial(
        pl.kernel,
        mesh=mesh,
        compiler_params=pltpu.CompilerParams(use_tc_tiling_on_sc=False),
        out_type=jax.ShapeDtypeStruct((E, 32), _F32),
        scratch_types=(
            [pltpu.VMEM((NBT, KB), jnp.int32)] * 2
            + [pltpu.VMEM((KB, 32), _F32)] * (4 * _NBUF)
            + [pltpu.SemaphoreType.DMA] * (2 * _NBUF)
        ),
    )
    def k(hs_hbm, hd_hbm, src_hbm, dst_hbm, c_hbm, t_hbm, siAll, diAll, *rs):
        bS = rs[0:_NBUF]
        bD = rs[_NBUF:2 * _NBUF]
        bC = rs[2 * _NBUF:3 * _NBUF]
        bT = rs[3 * _NBUF:4 * _NBUF]
        gsem = rs[4 * _NBUF:5 * _NBUF]
        osem = rs[5 * _NBUF:6 * _NBUF]
        wid = lax.axis_index("s") * 2 + lax.axis_index("c")
        nb = jnp.minimum(NBT, NB - wid * NBT)
        ebase = wid * (NBT * KB)
        pltpu.sync_copy(src_hbm.at[pl.ds(wid * NBT, NBT)], siAll)
        pltpu.sync_copy(dst_hbm.at[pl.ds(wid * NBT, NBT)], diAll)

        def issue(j, s):
            @pl.when(j < nb)
            def _():
                pltpu.async_copy(hs_hbm.at[siAll.at[j]], bS[s], gsem[s])
                pltpu.async_copy(hd_hbm.at[diAll.at[j]], bD[s], gsem[s])
                pltpu.async_copy(c_hbm.at[pl.ds(ebase + j * KB, KB)], bC[s], gsem[s])

        def step(j, s):
            @pl.when(j < nb)
            def _():
                @pl.when(j >= _NBUF)
                def _w():
                    pltpu.make_async_copy(c_hbm.at[pl.ds(0, KB)], bT[s], osem[s]).wait()
                for dstb in (bS[s], bD[s], bC[s]):
                    pltpu.make_async_copy(c_hbm.at[pl.ds(0, KB)], dstb, gsem[s]).wait()

                def row(r, carry2):
                    for half in range(2):
                        sl = pl.ds(half * 16, 16)
                        v = bC[s][r, sl] + bS[s][r, sl] + bD[s][r, sl]
                        bT[s][r, sl] = v / (1.0 + jnp.exp(-v))
                    return carry2

                lax.fori_loop(0, KB, row, 0, unroll=4)
                pltpu.async_copy(bT[s], t_hbm.at[pl.ds(ebase + j * KB, KB)], osem[s])
                issue(j + _NBUF, s)

        for s in range(_NBUF):
            issue(s, s)

        def outer(g, carry):
            for s in range(_NBUF):
                step(g * _NBUF + s, s)
            return carry

        lax.fori_loop(0, -(-NBT // _NBUF), outer, 0)
        for s in range(_NBUF):
            pltpu.make_async_copy(c_hbm.at[pl.ds(0, KB)], bT[s], osem[s]).wait()

    return k(hs, hd, src2, dst2, c)


def _sc_scatter(t, dst2, z32):
    E = t.shape[0]
    Nn = z32.shape[0]
    KB = dst2.shape[1]
    NB = E // KB
    NBT = dst2.shape[0] // _NW
    mesh = plsc.VectorSubcoreMesh(core_axis_name="c", subcore_axis_name="s")

    @functools.partial(
        pl.kernel,
        mesh=mesh,
        compiler_params=pltpu.CompilerParams(use_tc_tiling_on_sc=False),
        out_type=jax.ShapeDtypeStruct((2, Nn, 32), _F32),
        scratch_types=(
            [pltpu.VMEM((KB,), jnp.int32)] * _NBUF
            + [pltpu.VMEM((KB, 32), _F32)] * _NBUF
            + [pltpu.VMEM_SHARED((Nn, 32), _F32)]
            + [pltpu.SemaphoreType.DMA] * _NBUF
        ),
    )
    def k(t_hbm, dst_hbm, z32_hbm, S_hbm, *rs):
        bI = rs[0:_NBUF]
        bT = rs[_NBUF:2 * _NBUF]
        table = rs[2 * _NBUF]
        tsem = rs[2 * _NBUF + 1:3 * _NBUF + 1]
        cid = lax.axis_index("c")
        sid = lax.axis_index("s")
        wid = sid * 2 + cid
        nb = jnp.minimum(NBT, NB - wid * NBT)
        ebase = wid * (NBT * KB)

        @pl.when(sid == 0)
        def _init():
            pltpu.sync_copy(z32_hbm, table)

        plsc.subcore_barrier()

        def issue(j, s):
            @pl.when(j < nb)
            def _():
                pltpu.async_copy(dst_hbm.at[wid * NBT + j], bI[s], tsem[s])
                pltpu.async_copy(t_hbm.at[pl.ds(ebase + j * KB, KB)], bT[s], tsem[s])

        def step(j, s):
            @pl.when(j < nb)
            def _():
                pltpu.make_async_copy(dst_hbm.at[0], bI[s], tsem[s]).wait()
                pltpu.make_async_copy(t_hbm.at[pl.ds(0, KB)], bT[s], tsem[s]).wait()
                pltpu.sync_copy(bT[s], table.at[bI[s]], add=True)
                issue(j + _NBUF, s)

        for s in range(_NBUF):
            issue(s, s)

        def outer(g, carry):
            for s in range(_NBUF):
                step(g * _NBUF + s, s)
            return carry

        lax.fori_loop(0, -(-NBT // _NBUF), outer, 0)
        plsc.subcore_barrier()

        @pl.when(sid == 0)
        def _out():
            pltpu.sync_copy(table, S_hbm.at[cid])

    return k(t, dst2, z32)


def _sc_edge_step(hs, hd, src2, dst2, c, z32, emit_t):
    E = c.shape[0]
    Nn = z32.shape[0]
    KB = src2.shape[1]
    NB = E // KB
    NBT = src2.shape[0] // _NW
    nbuf = 2
    mesh = plsc.VectorSubcoreMesh(core_axis_name="c", subcore_axis_name="s")
    nper = 6 if emit_t else 5
    out_type = (jax.ShapeDtypeStruct((2, Nn, 32), _F32),)
    if emit_t:
        out_type = out_type + (jax.ShapeDtypeStruct((E, 32), _F32),)

    scratch = []
    for _s in range(nbuf):
        scratch += [pltpu.VMEM((KB,), jnp.int32)] * 2
        scratch += [pltpu.VMEM((KB, 32), _F32)] * (nper - 2)
    scratch += [pltpu.VMEM_SHARED((Nn, 32), _F32)]
    scratch += [pltpu.SemaphoreType.DMA] * (3 * nbuf)

    @functools.partial(
        pl.kernel,
        mesh=mesh,
        compiler_params=pltpu.CompilerParams(use_tc_tiling_on_sc=False),
        out_type=out_type,
        scratch_types=scratch,
    )
    def k(hs_hbm, hd_hbm, src_hbm, dst_hbm, c_hbm, z32_hbm, *rest):
        if emit_t:
            S_hbm, t_hbm = rest[0], rest[1]
            rs = rest[2:]
        else:
            S_hbm = rest[0]
            rs = rest[1:]
        slots = [rs[i * nper:(i + 1) * nper] for i in range(nbuf)]
        table = rs[nbuf * nper]
        sems = rs[nbuf * nper + 1:]
        lsem = sems[0:nbuf]
        gsem = sems[nbuf:2 * nbuf]
        osem = sems[2 * nbuf:3 * nbuf]
        cid = lax.axis_index("c")
        sid = lax.axis_index("s")
        wid = sid * 2 + cid
        nb = jnp.minimum(NBT, NB - wid * NBT)
        ebase = wid * (NBT * KB)

        @pl.when(sid == 0)
        def _init():
            pltpu.sync_copy(z32_hbm, table)

        plsc.subcore_barrier()

        def stage_a(j, s):
            @pl.when(j < nb)
            def _():
                sl = slots[s]
                pltpu.async_copy(src_hbm.at[wid * NBT + j], sl[0], lsem[s])
                pltpu.async_copy(dst_hbm.at[wid * NBT + j], sl[1], lsem[s])
                pltpu.async_copy(c_hbm.at[pl.ds(ebase + j * KB, KB)], sl[4], lsem[s])

        def stage_b(j, s):
            @pl.when(j < nb)
            def _():
                sl = slots[s]
                pltpu.make_async_copy(src_hbm.at[0], sl[0], lsem[s]).wait()
                pltpu.make_async_copy(src_hbm.at[0], sl[1], lsem[s]).wait()
                pltpu.make_async_copy(c_hbm.at[pl.ds(0, KB)], sl[4], lsem[s]).wait()
                pltpu.async_copy(hs_hbm.at[sl[0]], sl[2], gsem[s])
                pltpu.async_copy(hd_hbm.at[sl[1]], sl[3], gsem[s])

        def stage_c(j, s):
            @pl.when(j < nb)
            def _():
                sl = slots[s]
                pltpu.make_async_copy(c_hbm.at[pl.ds(0, KB)], sl[2], gsem[s]).wait()
                pltpu.make_async_copy(c_hbm.at[pl.ds(0, KB)], sl[3], gsem[s]).wait()
                if emit_t:
                    @pl.when(j >= nbuf)
                    def _w():
                        pltpu.make_async_copy(c_hbm.at[pl.ds(0, KB)], sl[5], osem[s]).wait()
                dstbuf = sl[5] if emit_t else sl[4]

                def row(r, carry2):
                    for half in range(2):
                        cs = pl.ds(half * 16, 16)
                        v = sl[4][r, cs] + sl[2][r, cs] + sl[3][r, cs]
                        dstbuf[r, cs] = v / (1.0 + jnp.exp(-v))
                    return carry2

                lax.fori_loop(0, KB, row, 0, unroll=4)
                pltpu.sync_copy(dstbuf, table.at[sl[1]], add=True)
                if emit_t:
                    pltpu.async_copy(dstbuf, t_hbm.at[pl.ds(ebase + j * KB, KB)],
                                     osem[s])

        look = nbuf - 1
        for s in range(nbuf):
            stage_a(s, s)
        for l in range(look):
            stage_b(l, l % nbuf)

        def outer(g, carry):
            for s in range(nbuf):
                j = g * nbuf + s
                stage_b(j + look, (s + look) % nbuf)
                stage_c(j, s)
                stage_a(j + nbuf, s)
            return carry

        lax.fori_loop(0, -(-NBT // nbuf), outer, 0)
        if emit_t:
            for s in range(nbuf):
                pltpu.make_async_copy(c_hbm.at[pl.ds(0, KB)], slots[s][5],
                                      osem[s]).wait()
        plsc.subcore_barrier()

        @pl.when(sid == 0)
        def _out():
            pltpu.sync_copy(table, S_hbm.at[cid])

    return k(hs, hd, src2, dst2, c, z32)


def _sc_degree(dst2, z4, o4, E):
    Nn = z4.shape[0]
    KB = dst2.shape[1]
    NB = E // KB
    NBT = dst2.shape[0] // _NW
    mesh = plsc.VectorSubcoreMesh(core_axis_name="c", subcore_axis_name="s")

    @functools.partial(
        pl.kernel,
        mesh=mesh,
        compiler_params=pltpu.CompilerParams(use_tc_tiling_on_sc=False),
        out_type=jax.ShapeDtypeStruct((2, Nn, 4), _F32),
        scratch_types=[
            pltpu.VMEM((NBT, KB), jnp.int32),
            pltpu.VMEM((KB, 4), _F32),
            pltpu.VMEM_SHARED((Nn, 4), _F32),
        ],
    )
    def k(dst_hbm, z4_hbm, o4_hbm, D_hbm, diAll, onev, degt):
        cid = lax.axis_index("c")
        sid = lax.axis_index("s")
        wid = sid * 2 + cid
        nb = jnp.minimum(NBT, NB - wid * NBT)

        @pl.when(sid == 0)
        def _init():
            pltpu.sync_copy(z4_hbm, degt)

        pltpu.sync_copy(o4_hbm, onev)
        pltpu.sync_copy(dst_hbm.at[pl.ds(wid * NBT, NBT)], diAll)
        plsc.subcore_barrier()

        def blk(j, carry):
            pltpu.sync_copy(onev, degt.at[diAll.at[j]], add=True)
            return carry

        lax.fori_loop(0, nb, blk, 0)
        plsc.subcore_barrier()

        @pl.when(sid == 0)
        def _out():
            pltpu.sync_copy(degt, D_hbm.at[cid])

    return k(dst2, z4, o4)


def kernel(x, edge_index, edge_attr, params):
    x2 = x[0]
    ea2 = edge_attr[0]
    src = edge_index[0]
    dst = edge_index[1]
    Nn = x2.shape[0]

    (W0, b0), (W1, b1) = params["enc_node"]
    (V0, vb0), (V1, vb1) = params["enc_edge"]
    (D0, e0), (D1, e1) = params["dec"]
    st = []
    for s in range(2):
        (A1, a1), (B1, d1) = params["steps"][s]["edge"]
        (P1, p1), (Q1, q1) = params["steps"][s]["node"]
        st.append(dict(Ae=A1[0:32], As=A1[32:64], Ad=A1[64:96], a=a1, B=B1,
                       d=d1, Ph=P1[0:32], Pa=P1[32:64], p=p1, Q=Q1, q=q1))

    r1 = lambda v: v.reshape(1, -1)
    mm = functools.partial(jnp.matmul)
    G0 = mm(V1, st[0]["Ae"])
    g0 = mm(vb1, st[0]["Ae"]) + st[0]["a"]
    G1 = mm(V1, st[1]["Ae"])
    g1 = mm(vb1 + st[0]["d"], st[1]["Ae"]) + st[1]["a"]
    M = mm(st[0]["B"], st[1]["Ae"])

    c0, c1 = _tc_edge_enc(ea2, V0, r1(vb0), G0, r1(g0), G1, r1(g1))
    h0, hs0, hd0 = _tc_node_enc(x2, W0, r1(b0), W1, r1(b1), st[0]["As"], st[0]["Ad"])

    z32 = jnp.zeros((Nn, 32), _F32)
    z4 = jnp.zeros((Nn, 4), _F32)

    KB = 128
    E = src.shape[0]
    NB = E // KB
    NBT = -(-NB // _NW)
    padlen = _NW * NBT * KB - E
    zpad = jnp.zeros((padlen,), jnp.int32)
    src2 = jnp.concatenate([src, zpad]).reshape(_NW * NBT, KB)
    dst2 = jnp.concatenate([dst, zpad]).reshape(_NW * NBT, KB)
    o4 = jnp.ones((KB, 4), _F32)

    Dp0 = _sc_degree(dst2, z4, o4, E)
    t0 = _sc_gather_silu(hs0, hd0, src2, dst2, c0)
    Sp0 = _sc_scatter(t0, dst2, z32)
    c1p = _tc_cupdate(c1, t0, M)
    h1, hs1, hd1 = _tc_node_update(
        h0, Sp0[0], Sp0[1], Dp0[0], Dp0[1], st[0]["B"], r1(st[0]["d"]),
        st[0]["Ph"], st[0]["Pa"], r1(st[0]["p"]), st[0]["Q"], r1(st[0]["q"]),
        st[1]["As"], st[1]["Ad"])

    (Sp1,) = _sc_edge_step(hs1, hd1, src2, dst2, c1p, z32, False)
    y = _tc_node_dec(
        h1, Sp1[0], Sp1[1], Dp0[0], Dp0[1], st[1]["B"], r1(st[1]["d"]),
        st[1]["Ph"], st[1]["Pa"], r1(st[1]["p"]), st[1]["Q"], r1(st[1]["q"]),
        D0, r1(e0), D1, r1(e1))
    return y[None]


# speedup vs baseline: 1.0628x; 1.0173x over previous
import functools

import jax
import jax.numpy as jnp
from jax import lax
from jax.experimental import pallas as pl
from jax.experimental.pallas import tpu as pltpu
from jax.experimental.pallas import tpu_sc as plsc

_F32 = jnp.float32
_NW = 32
_K = 128
_NBUF = 3


def _w(shape):
    nd = len(shape)
    return pl.BlockSpec(shape, lambda i, _nd=nd: (0,) * nd)


def _row(blk, d):
    return pl.BlockSpec((blk, d), lambda i: (i, 0))


def _tc_edge_enc(ea, V0, vb0, G0, g0, G1, g1):
    E = ea.shape[0]
    BLK = 2000
    def body(ear, V0r, vb0r, G0r, g0r, G1r, g1r, c0r, c1r):
        z = jnp.dot(ear[...], V0r[...], preferred_element_type=_F32) + vb0r[...]
        z = z * jax.nn.sigmoid(z)
        c0r[...] = jnp.dot(z, G0r[...], preferred_element_type=_F32) + g0r[...]
        c1r[...] = jnp.dot(z, G1r[...], preferred_element_type=_F32) + g1r[...]
    return pl.pallas_call(
        body,
        grid=(E // BLK,),
        in_specs=[_row(BLK, 4), _w((4, 32)), _w((1, 32)), _w((32, 32)),
                  _w((1, 32)), _w((32, 32)), _w((1, 32))],
        out_specs=[_row(BLK, 32), _row(BLK, 32)],
        out_shape=[jax.ShapeDtypeStruct((E, 32), _F32)] * 2,
    )(ea, V0, vb0, G0, g0, G1, g1)


def _tc_node_enc(x2, W0, b0, W1, b1, As, Ad):
    Nn = x2.shape[0]
    BLK = 2000
    def body(xr, W0r, b0r, W1r, b1r, Asr, Adr, hr, hsr, hdr):
        z = jnp.dot(xr[...], W0r[...], preferred_element_type=_F32) + b0r[...]
        z = z * jax.nn.sigmoid(z)
        h = jnp.dot(z, W1r[...], preferred_element_type=_F32) + b1r[...]
        hr[...] = h
        hsr[...] = jnp.dot(h, Asr[...], preferred_element_type=_F32)
        hdr[...] = jnp.dot(h, Adr[...], preferred_element_type=_F32)
    return pl.pallas_call(
        body,
        grid=(Nn // BLK,),
        in_specs=[_row(BLK, 16), _w((16, 32)), _w((1, 32)), _w((32, 32)),
                  _w((1, 32)), _w((32, 32)), _w((32, 32))],
        out_specs=[_row(BLK, 32)] * 3,
        out_shape=[jax.ShapeDtypeStruct((Nn, 32), _F32)] * 3,
    )(x2, W0, b0, W1, b1, As, Ad)


def _tc_cupdate(c1, t0, M):
    E = c1.shape[0]
    BLK = 2000
    def body(c1r, t0r, Mr, outr):
        outr[...] = c1r[...] + jnp.dot(t0r[...], Mr[...], preferred_element_type=_F32)
    return pl.pallas_call(
        body,
        grid=(E // BLK,),
        in_specs=[_row(BLK, 32), _row(BLK, 32), _w((32, 32))],
        out_specs=_row(BLK, 32),
        out_shape=jax.ShapeDtypeStruct((E, 32), _F32),
    )(c1, t0, M)


def _tc_node_update(h, Sa, Sb, dga, dgb, B, d, Ph, Pa, p, Q, q, As, Ad):
    Nn = h.shape[0]
    BLK = 2000
    def body(hr, Sar, Sbr, dgar, dgbr, Br, dr, Phr, Par, pr, Qr, qr, Asr, Adr,
             h1r, hsr, hdr):
        deg = dgar[:, 0:1] + dgbr[:, 0:1]
        agg = jnp.dot(Sar[...] + Sbr[...], Br[...], preferred_element_type=_F32) \
            + deg * dr[...]
        u = jnp.dot(hr[...], Phr[...], preferred_element_type=_F32) \
            + jnp.dot(agg, Par[...], preferred_element_type=_F32) + pr[...]
        u = u * jax.nn.sigmoid(u)
        h1 = hr[...] + jnp.dot(u, Qr[...], preferred_element_type=_F32) + qr[...]
        h1r[...] = h1
        hsr[...] = jnp.dot(h1, Asr[...], preferred_element_type=_F32)
        hdr[...] = jnp.dot(h1, Adr[...], preferred_element_type=_F32)
    return pl.pallas_call(
        body,
        grid=(Nn // BLK,),
        in_specs=[_row(BLK, 32), _row(BLK, 32), _row(BLK, 32), _row(BLK, 4),
                  _row(BLK, 4), _w((32, 32)), _w((1, 32)), _w((32, 32)),
                  _w((32, 32)), _w((1, 32)), _w((32, 32)), _w((1, 32)),
                  _w((32, 32)), _w((32, 32))],
        out_specs=[_row(BLK, 32)] * 3,
        out_shape=[jax.ShapeDtypeStruct((Nn, 32), _F32)] * 3,
    )(h, Sa, Sb, dga, dgb, B, d, Ph, Pa, p, Q, q, As, Ad)


def _tc_node_dec(h, Sa, Sb, dga, dgb, B, d, Ph, Pa, p, Q, q, D0, e0, D1, e1):
    Nn = h.shape[0]
    BLK = 2000
    def body(hr, Sar, Sbr, dgar, dgbr, Br, dr, Phr, Par, pr, Qr, qr,
             D0r, e0r, D1r, e1r, yr):
        deg = dgar[:, 0:1] + dgbr[:, 0:1]
        agg = jnp.dot(Sar[...] + Sbr[...], Br[...], preferred_element_type=_F32) \
            + deg * dr[...]
        u = jnp.dot(hr[...], Phr[...], preferred_element_type=_F32) \
            + jnp.dot(agg, Par[...], preferred_element_type=_F32) + pr[...]
        u = u * jax.nn.sigmoid(u)
        h2 = hr[...] + jnp.dot(u, Qr[...], preferred_element_type=_F32) + qr[...]
        z = jnp.dot(h2, D0r[...], preferred_element_type=_F32) + e0r[...]
        z = z * jax.nn.sigmoid(z)
        yr[...] = jnp.dot(z, D1r[...], preferred_element_type=_F32) + e1r[...]
    return pl.pallas_call(
        body,
        grid=(Nn // BLK,),
        in_specs=[_row(BLK, 32), _row(BLK, 32), _row(BLK, 32), _row(BLK, 4),
                  _row(BLK, 4), _w((32, 32)), _w((1, 32)), _w((32, 32)),
                  _w((32, 32)), _w((1, 32)), _w((32, 32)), _w((1, 32)),
                  _w((32, 32)), _w((1, 32)), _w((32, 16)), _w((1, 16))],
        out_specs=_row(BLK, 16),
        out_shape=jax.ShapeDtypeStruct((Nn, 16), _F32),
    )(h, Sa, Sb, dga, dgb, B, d, Ph, Pa, p, Q, q, D0, e0, D1, e1)


def _sc_gather_silu(hs, hd, src2, dst2, c):
    E = c.shape[0]
    KB = src2.shape[1]
    NB = E // KB
    NBT = src2.shape[0] // _NW
    mesh = plsc.VectorSubcoreMesh(core_axis_name="c", subcore_axis_name="s")

    @functools.partial(
        pl.kernel,
        mesh=mesh,
        compiler_params=pltpu.CompilerParams(use_tc_tiling_on_sc=False, skip_device_barrier=True),
        out_type=jax.ShapeDtypeStruct((E, 32), _F32),
        scratch_types=(
            [pltpu.VMEM((NBT, KB), jnp.int32)] * 2
            + [pltpu.VMEM((KB, 32), _F32)] * (4 * _NBUF)
            + [pltpu.SemaphoreType.DMA] * (2 * _NBUF)
        ),
    )
    def k(hs_hbm, hd_hbm, src_hbm, dst_hbm, c_hbm, t_hbm, siAll, diAll, *rs):
        bS = rs[0:_NBUF]
        bD = rs[_NBUF:2 * _NBUF]
        bC = rs[2 * _NBUF:3 * _NBUF]
        bT = rs[3 * _NBUF:4 * _NBUF]
        gsem = rs[4 * _NBUF:5 * _NBUF]
        osem = rs[5 * _NBUF:6 * _NBUF]
        wid = lax.axis_index("s") * 2 + lax.axis_index("c")
        nb = jnp.minimum(NBT, NB - wid * NBT)
        ebase = wid * (NBT * KB)
        pltpu.sync_copy(src_hbm.at[pl.ds(wid * NBT, NBT)], siAll)
        pltpu.sync_copy(dst_hbm.at[pl.ds(wid * NBT, NBT)], diAll)

        def issue(j, s):
            @pl.when(j < nb)
            def _():
                pltpu.async_copy(hs_hbm.at[siAll.at[j]], bS[s], gsem[s])
                pltpu.async_copy(hd_hbm.at[diAll.at[j]], bD[s], gsem[s])
                pltpu.async_copy(c_hbm.at[pl.ds(ebase + j * KB, KB)], bC[s], gsem[s])

        def step(j, s):
            @pl.when(j < nb)
            def _():
                @pl.when(j >= _NBUF)
                def _w():
                    pltpu.make_async_copy(c_hbm.at[pl.ds(0, KB)], bT[s], osem[s]).wait()
                for dstb in (bS[s], bD[s], bC[s]):
                    pltpu.make_async_copy(c_hbm.at[pl.ds(0, KB)], dstb, gsem[s]).wait()

                def row(r, carry2):
                    for half in range(2):
                        sl = pl.ds(half * 16, 16)
                        v = bC[s][r, sl] + bS[s][r, sl] + bD[s][r, sl]
                        bT[s][r, sl] = v / (1.0 + jnp.exp(-v))
                    return carry2

                lax.fori_loop(0, KB, row, 0, unroll=4)
                pltpu.async_copy(bT[s], t_hbm.at[pl.ds(ebase + j * KB, KB)], osem[s])
                issue(j + _NBUF, s)

        for s in range(_NBUF):
            issue(s, s)

        def outer(g, carry):
            for s in range(_NBUF):
                step(g * _NBUF + s, s)
            return carry

        lax.fori_loop(0, -(-NBT // _NBUF), outer, 0)
        for s in range(_NBUF):
            pltpu.make_async_copy(c_hbm.at[pl.ds(0, KB)], bT[s], osem[s]).wait()

    return k(hs, hd, src2, dst2, c)


def _sc_scatter(t, dst2, z32):
    E = t.shape[0]
    Nn = z32.shape[0]
    KB = dst2.shape[1]
    NB = E // KB
    NBT = dst2.shape[0] // _NW
    mesh = plsc.VectorSubcoreMesh(core_axis_name="c", subcore_axis_name="s")

    @functools.partial(
        pl.kernel,
        mesh=mesh,
        compiler_params=pltpu.CompilerParams(use_tc_tiling_on_sc=False, skip_device_barrier=True),
        out_type=jax.ShapeDtypeStruct((2, Nn, 32), _F32),
        scratch_types=(
            [pltpu.VMEM((KB,), jnp.int32)] * _NBUF
            + [pltpu.VMEM((KB, 32), _F32)] * _NBUF
            + [pltpu.VMEM_SHARED((Nn, 32), _F32)]
            + [pltpu.SemaphoreType.DMA] * _NBUF
        ),
    )
    def k(t_hbm, dst_hbm, z32_hbm, S_hbm, *rs):
        bI = rs[0:_NBUF]
        bT = rs[_NBUF:2 * _NBUF]
        table = rs[2 * _NBUF]
        tsem = rs[2 * _NBUF + 1:3 * _NBUF + 1]
        cid = lax.axis_index("c")
        sid = lax.axis_index("s")
        wid = sid * 2 + cid
        nb = jnp.minimum(NBT, NB - wid * NBT)
        ebase = wid * (NBT * KB)

        @pl.when(sid == 0)
        def _init():
            pltpu.sync_copy(z32_hbm, table)

        plsc.subcore_barrier()

        def issue(j, s):
            @pl.when(j < nb)
            def _():
                pltpu.async_copy(dst_hbm.at[wid * NBT + j], bI[s], tsem[s])
                pltpu.async_copy(t_hbm.at[pl.ds(ebase + j * KB, KB)], bT[s], tsem[s])

        def step(j, s):
            @pl.when(j < nb)
            def _():
                pltpu.make_async_copy(dst_hbm.at[0], bI[s], tsem[s]).wait()
                pltpu.make_async_copy(t_hbm.at[pl.ds(0, KB)], bT[s], tsem[s]).wait()
                pltpu.sync_copy(bT[s], table.at[bI[s]], add=True)
                issue(j + _NBUF, s)

        for s in range(_NBUF):
            issue(s, s)

        def outer(g, carry):
            for s in range(_NBUF):
                step(g * _NBUF + s, s)
            return carry

        lax.fori_loop(0, -(-NBT // _NBUF), outer, 0)
        plsc.subcore_barrier()

        @pl.when(sid == 0)
        def _out():
            pltpu.sync_copy(table, S_hbm.at[cid])

    return k(t, dst2, z32)


def _sc_edge_step(hs, hd, src2, dst2, c, z32, emit_t):
    E = c.shape[0]
    Nn = z32.shape[0]
    KB = src2.shape[1]
    NB = E // KB
    NBT = src2.shape[0] // _NW
    nbuf = 2
    mesh = plsc.VectorSubcoreMesh(core_axis_name="c", subcore_axis_name="s")
    nper = 6 if emit_t else 5
    out_type = (jax.ShapeDtypeStruct((2, Nn, 32), _F32),)
    if emit_t:
        out_type = out_type + (jax.ShapeDtypeStruct((E, 32), _F32),)

    scratch = []
    for _s in range(nbuf):
        scratch += [pltpu.VMEM((KB,), jnp.int32)] * 2
        scratch += [pltpu.VMEM((KB, 32), _F32)] * (nper - 2)
    scratch += [pltpu.VMEM_SHARED((Nn, 32), _F32)]
    scratch += [pltpu.SemaphoreType.DMA] * (3 * nbuf)

    @functools.partial(
        pl.kernel,
        mesh=mesh,
        compiler_params=pltpu.CompilerParams(use_tc_tiling_on_sc=False, skip_device_barrier=True),
        out_type=out_type,
        scratch_types=scratch,
    )
    def k(hs_hbm, hd_hbm, src_hbm, dst_hbm, c_hbm, z32_hbm, *rest):
        if emit_t:
            S_hbm, t_hbm = rest[0], rest[1]
            rs = rest[2:]
        else:
            S_hbm = rest[0]
            rs = rest[1:]
        slots = [rs[i * nper:(i + 1) * nper] for i in range(nbuf)]
        table = rs[nbuf * nper]
        sems = rs[nbuf * nper + 1:]
        lsem = sems[0:nbuf]
        gsem = sems[nbuf:2 * nbuf]
        osem = sems[2 * nbuf:3 * nbuf]
        cid = lax.axis_index("c")
        sid = lax.axis_index("s")
        wid = sid * 2 + cid
        nb = jnp.minimum(NBT, NB - wid * NBT)
        ebase = wid * (NBT * KB)

        @pl.when(sid == 0)
        def _init():
            pltpu.sync_copy(z32_hbm, table)

        plsc.subcore_barrier()

        def stage_a(j, s):
            @pl.when(j < nb)
            def _():
                sl = slots[s]
                pltpu.async_copy(src_hbm.at[wid * NBT + j], sl[0], lsem[s])
                pltpu.async_copy(dst_hbm.at[wid * NBT + j], sl[1], lsem[s])
                pltpu.async_copy(c_hbm.at[pl.ds(ebase + j * KB, KB)], sl[4], lsem[s])

        def stage_b(j, s):
            @pl.when(j < nb)
            def _():
                sl = slots[s]
                pltpu.make_async_copy(src_hbm.at[0], sl[0], lsem[s]).wait()
                pltpu.make_async_copy(src_hbm.at[0], sl[1], lsem[s]).wait()
                pltpu.make_async_copy(c_hbm.at[pl.ds(0, KB)], sl[4], lsem[s]).wait()
                pltpu.async_copy(hs_hbm.at[sl[0]], sl[2], gsem[s])
                pltpu.async_copy(hd_hbm.at[sl[1]], sl[3], gsem[s])

        def stage_c(j, s):
            @pl.when(j < nb)
            def _():
                sl = slots[s]
                pltpu.make_async_copy(c_hbm.at[pl.ds(0, KB)], sl[2], gsem[s]).wait()
                pltpu.make_async_copy(c_hbm.at[pl.ds(0, KB)], sl[3], gsem[s]).wait()
                if emit_t:
                    @pl.when(j >= nbuf)
                    def _w():
                        pltpu.make_async_copy(c_hbm.at[pl.ds(0, KB)], sl[5], osem[s]).wait()
                dstbuf = sl[5] if emit_t else sl[4]

                def row(r, carry2):
                    for half in range(2):
                        cs = pl.ds(half * 16, 16)
                        v = sl[4][r, cs] + sl[2][r, cs] + sl[3][r, cs]
                        dstbuf[r, cs] = v / (1.0 + jnp.exp(-v))
                    return carry2

                lax.fori_loop(0, KB, row, 0, unroll=4)
                pltpu.sync_copy(dstbuf, table.at[sl[1]], add=True)
                if emit_t:
                    pltpu.async_copy(dstbuf, t_hbm.at[pl.ds(ebase + j * KB, KB)],
                                     osem[s])

        look = nbuf - 1
        for s in range(nbuf):
            stage_a(s, s)
        for l in range(look):
            stage_b(l, l % nbuf)

        def outer(g, carry):
            for s in range(nbuf):
                j = g * nbuf + s
                stage_b(j + look, (s + look) % nbuf)
                stage_c(j, s)
                stage_a(j + nbuf, s)
            return carry

        lax.fori_loop(0, -(-NBT // nbuf), outer, 0)
        if emit_t:
            for s in range(nbuf):
                pltpu.make_async_copy(c_hbm.at[pl.ds(0, KB)], slots[s][5],
                                      osem[s]).wait()
        plsc.subcore_barrier()

        @pl.when(sid == 0)
        def _out():
            pltpu.sync_copy(table, S_hbm.at[cid])

    return k(hs, hd, src2, dst2, c, z32)


def _sc_degree(dst2, z4, o4, E):
    Nn = z4.shape[0]
    KB = dst2.shape[1]
    NB = E // KB
    NBT = dst2.shape[0] // _NW
    mesh = plsc.VectorSubcoreMesh(core_axis_name="c", subcore_axis_name="s")

    @functools.partial(
        pl.kernel,
        mesh=mesh,
        compiler_params=pltpu.CompilerParams(use_tc_tiling_on_sc=False, skip_device_barrier=True),
        out_type=jax.ShapeDtypeStruct((2, Nn, 4), _F32),
        scratch_types=[
            pltpu.VMEM((NBT, KB), jnp.int32),
            pltpu.VMEM((KB, 4), _F32),
            pltpu.VMEM_SHARED((Nn, 4), _F32),
        ],
    )
    def k(dst_hbm, z4_hbm, o4_hbm, D_hbm, diAll, onev, degt):
        cid = lax.axis_index("c")
        sid = lax.axis_index("s")
        wid = sid * 2 + cid
        nb = jnp.minimum(NBT, NB - wid * NBT)

        @pl.when(sid == 0)
        def _init():
            pltpu.sync_copy(z4_hbm, degt)

        pltpu.sync_copy(o4_hbm, onev)
        pltpu.sync_copy(dst_hbm.at[pl.ds(wid * NBT, NBT)], diAll)
        plsc.subcore_barrier()

        def blk(j, carry):
            pltpu.sync_copy(onev, degt.at[diAll.at[j]], add=True)
            return carry

        lax.fori_loop(0, nb, blk, 0)
        plsc.subcore_barrier()

        @pl.when(sid == 0)
        def _out():
            pltpu.sync_copy(degt, D_hbm.at[cid])

    return k(dst2, z4, o4)


def kernel(x, edge_index, edge_attr, params):
    x2 = x[0]
    ea2 = edge_attr[0]
    src = edge_index[0]
    dst = edge_index[1]
    Nn = x2.shape[0]

    (W0, b0), (W1, b1) = params["enc_node"]
    (V0, vb0), (V1, vb1) = params["enc_edge"]
    (D0, e0), (D1, e1) = params["dec"]
    st = []
    for s in range(2):
        (A1, a1), (B1, d1) = params["steps"][s]["edge"]
        (P1, p1), (Q1, q1) = params["steps"][s]["node"]
        st.append(dict(Ae=A1[0:32], As=A1[32:64], Ad=A1[64:96], a=a1, B=B1,
                       d=d1, Ph=P1[0:32], Pa=P1[32:64], p=p1, Q=Q1, q=q1))

    r1 = lambda v: v.reshape(1, -1)
    mm = functools.partial(jnp.matmul)
    G0 = mm(V1, st[0]["Ae"])
    g0 = mm(vb1, st[0]["Ae"]) + st[0]["a"]
    G1 = mm(V1, st[1]["Ae"])
    g1 = mm(vb1 + st[0]["d"], st[1]["Ae"]) + st[1]["a"]
    M = mm(st[0]["B"], st[1]["Ae"])

    c0, c1 = _tc_edge_enc(ea2, V0, r1(vb0), G0, r1(g0), G1, r1(g1))
    h0, hs0, hd0 = _tc_node_enc(x2, W0, r1(b0), W1, r1(b1), st[0]["As"], st[0]["Ad"])

    z32 = jnp.zeros((Nn, 32), _F32)
    z4 = jnp.zeros((Nn, 4), _F32)

    KB = 128
    E = src.shape[0]
    NB = E // KB
    NBT = -(-NB // _NW)
    padlen = _NW * NBT * KB - E
    zpad = jnp.zeros((padlen,), jnp.int32)
    src2 = jnp.concatenate([src, zpad]).reshape(_NW * NBT, KB)
    dst2 = jnp.concatenate([dst, zpad]).reshape(_NW * NBT, KB)
    o4 = jnp.ones((KB, 4), _F32)

    Dp0 = _sc_degree(dst2, z4, o4, E)
    t0 = _sc_gather_silu(hs0, hd0, src2, dst2, c0)
    Sp0 = _sc_scatter(t0, dst2, z32)
    c1p = _tc_cupdate(c1, t0, M)
    h1, hs1, hd1 = _tc_node_update(
        h0, Sp0[0], Sp0[1], Dp0[0], Dp0[1], st[0]["B"], r1(st[0]["d"]),
        st[0]["Ph"], st[0]["Pa"], r1(st[0]["p"]), st[0]["Q"], r1(st[0]["q"]),
        st[1]["As"], st[1]["Ad"])

    t1 = _sc_gather_silu(hs1, hd1, src2, dst2, c1p)
    Sp1 = _sc_scatter(t1, dst2, z32)
    y = _tc_node_dec(
        h1, Sp1[0], Sp1[1], Dp0[0], Dp0[1], st[1]["B"], r1(st[1]["d"]),
        st[1]["Ph"], st[1]["Pa"], r1(st[1]["p"]), st[1]["Q"], r1(st[1]["q"]),
        D0, r1(e0), D1, r1(e1))
    return y[None]
